# Initial kernel scaffold; baseline (speedup 1.0000x reference)
import jax
import jax.numpy as jnp
from jax.experimental import pallas as pl


def kernel(current, previous, working_keys, working_values, working_protection, working_usage, working_age, semantic_keys, semantic_values, params):
    raise NotImplementedError("write your pallas kernel here")


# speedup vs baseline: 8.5393x; 8.5393x over previous
import functools

import jax
import jax.numpy as jnp
from jax import lax
from jax.experimental import pallas as pl
from jax.experimental.pallas import tpu as pltpu
from jax.experimental.pallas import tpu_sc as plsc

D = 1024
N = 32
S = 2048
K = 8
B = 1024

_PREC = lax.Precision.HIGHEST


def _dot(a, b):
    return lax.dot_general(a, b, (((1,), (0,)), ((), ())),
                           precision=_PREC, preferred_element_type=jnp.float32)


def _dot_t(a, b):
    return lax.dot_general(a, b, (((1,), (1,)), ((), ())),
                           precision=_PREC, preferred_element_type=jnp.float32)


_BB1 = 64


def _router_body(cur_ref, prev_ref, wk_ref, sk_ref, wq_ref,
                 semidx_ref, widx_ref, wsem_ref, wwork_ref):
    cur = cur_ref[...]
    prev = prev_ref[...]
    q = _dot(cur, wq_ref[pl.ds(0, D), :]) + _dot(prev, wq_ref[pl.ds(D, D), :])
    q = q / jnp.maximum(jnp.sqrt(jnp.sum(q * q, -1, keepdims=True)), 1e-6)
    sk = sk_ref[...]
    skn = jnp.maximum(jnp.sqrt(jnp.sum(sk * sk, -1, keepdims=True)), 1e-6)
    nsk = sk / skn
    ss = _dot_t(q, nsk)
    ws_list = []
    for n in range(N):
        wkn = wk_ref[:, n, :]
        dp = jnp.sum(q * wkn, -1, keepdims=True)
        nn = jnp.maximum(jnp.sqrt(jnp.sum(wkn * wkn, -1, keepdims=True)), 1e-6)
        ws_list.append(dp / nn)
    ws = jnp.concatenate(ws_list, -1)
    scores = jnp.concatenate([ws, ss], -1)
    T = N + S
    iota = lax.broadcasted_iota(jnp.int32, (_BB1, T), 1)
    s = scores
    tops, topi = [], []
    for _ in range(K):
        m = jnp.max(s, -1, keepdims=True)
        idx = jnp.min(jnp.where(s == m, iota, T), -1, keepdims=True)
        tops.append(m)
        topi.append(idx)
        s = jnp.where(iota == idx, -jnp.inf, s)
    top_s = jnp.concatenate(tops, -1)
    top_i = jnp.concatenate(topi, -1)
    e = jnp.exp(top_s - jnp.max(top_s, -1, keepdims=True))
    w = e / jnp.sum(e, -1, keepdims=True)
    row = pl.program_id(0) * _BB1 + lax.broadcasted_iota(jnp.int32, (_BB1, K), 0)
    semidx_ref[...] = jnp.clip(top_i - N, 0, S - 1)
    widx_ref[...] = row * N + jnp.clip(top_i, 0, N - 1)
    wsem_ref[...] = jnp.where(top_i >= N, w, 0.0)
    wwork_ref[...] = jnp.where(top_i < N, w, 0.0)


def _router(cur, prev, wk, sk, wq):
    grid = (B // _BB1,)
    return pl.pallas_call(
        _router_body,
        grid=grid,
        in_specs=[
            pl.BlockSpec((_BB1, D), lambda i: (i, 0)),
            pl.BlockSpec((_BB1, D), lambda i: (i, 0)),
            pl.BlockSpec((_BB1, N, D), lambda i: (i, 0, 0)),
            pl.BlockSpec((S, D), lambda i: (0, 0)),
            pl.BlockSpec((2 * D, D), lambda i: (0, 0)),
        ],
        out_specs=[
            pl.BlockSpec((_BB1, K), lambda i: (i, 0)),
            pl.BlockSpec((_BB1, K), lambda i: (i, 0)),
            pl.BlockSpec((_BB1, K), lambda i: (i, 0)),
            pl.BlockSpec((_BB1, K), lambda i: (i, 0)),
        ],
        out_shape=[
            jax.ShapeDtypeStruct((B, K), jnp.int32),
            jax.ShapeDtypeStruct((B, K), jnp.int32),
            jax.ShapeDtypeStruct((B, K), jnp.float32),
            jax.ShapeDtypeStruct((B, K), jnp.float32),
        ],
        compiler_params=pltpu.CompilerParams(
            dimension_semantics=("arbitrary",)),
    )(cur, prev, wk, sk, wq)


_ROWS = B * K
_NW = 32
_RPW = _ROWS // _NW
_CH = 32
_NCHUNK = _RPW // _CH


def _gather_rows(sem_idx_flat, widx_flat, sv, wv_flat):
    mesh = plsc.VectorSubcoreMesh(core_axis_name="c", subcore_axis_name="s")

    @functools.partial(
        pl.kernel,
        out_type=(jax.ShapeDtypeStruct((_ROWS, D), jnp.float32),
                  jax.ShapeDtypeStruct((_ROWS, D), jnp.float32)),
        mesh=mesh,
        scratch_types=[
            pltpu.VMEM((_CH,), jnp.int32),
            pltpu.VMEM((_CH, D), jnp.float32),
            pltpu.VMEM((_CH,), jnp.int32),
            pltpu.VMEM((_CH, D), jnp.float32),
            pltpu.SemaphoreType.DMA,
            pltpu.SemaphoreType.DMA,
        ],
    )
    def k(semidx_hbm, widx_hbm, sv_hbm, wvf_hbm, gsem_hbm, gwork_hbm,
          idx1_v, rows1_v, idx2_v, rows2_v, dsem1, dsem2):
        wid = lax.axis_index("s") * 2 + lax.axis_index("c")
        base = wid * _RPW

        def body(i, carry):
            off = base + i * _CH
            pltpu.sync_copy(semidx_hbm.at[pl.ds(off, _CH)], idx1_v)
            pltpu.sync_copy(widx_hbm.at[pl.ds(off, _CH)], idx2_v)
            c1 = pltpu.async_copy(sv_hbm.at[idx1_v], rows1_v, dsem1)
            c2 = pltpu.async_copy(wvf_hbm.at[idx2_v], rows2_v, dsem2)
            c1.wait()
            c2.wait()
            pltpu.sync_copy(rows1_v, gsem_hbm.at[pl.ds(off, _CH)])
            pltpu.sync_copy(rows2_v, gwork_hbm.at[pl.ds(off, _CH)])
            return carry

        lax.fori_loop(0, _NCHUNK, body, 0)

    return k(sem_idx_flat, widx_flat, sv, wv_flat)


_BB3 = 64


def _cell_body(cur_ref, prev_ref, gsem_ref, gwork_ref, wsem_ref, wwork_ref,
               wg_ref, wc_ref, wo_ref, bg_ref, bc_ref, bo_ref, lng_ref, lnb_ref,
               wkp_ref, wvp_ref, bkp_ref, bvp_ref, wsm_ref, bsm_ref,
               hidden_ref, ck_ref, cv_ref, hg_ref):
    cur = cur_ref[...]
    prev = prev_ref[...]
    mr = jnp.zeros_like(cur)
    for k in range(K):
        mr = mr + wsem_ref[:, k:k + 1] * gsem_ref[:, k, :]
        mr = mr + wwork_ref[:, k:k + 1] * gwork_ref[:, k, :]
    gate = jax.nn.sigmoid(_dot(cur, wg_ref[pl.ds(0, D), :])
                          + _dot(prev, wg_ref[pl.ds(D, D), :])
                          + _dot(mr, wg_ref[pl.ds(2 * D, D), :]) + bg_ref[...])
    cand = jnp.tanh(_dot(cur, wc_ref[pl.ds(0, D), :])
                    + _dot(prev, wc_ref[pl.ds(D, D), :])
                    + _dot(mr, wc_ref[pl.ds(2 * D, D), :]) + bc_ref[...])
    h = (1.0 - gate) * prev + gate * cand
    x = _dot(cur, wo_ref[pl.ds(0, D), :]) + _dot(h, wo_ref[pl.ds(D, D), :]) \
        + bo_ref[...] + cur
    mu = jnp.mean(x, -1, keepdims=True)
    xc = x - mu
    var = jnp.mean(xc * xc, -1, keepdims=True)
    hidden = xc / jnp.sqrt(var + 1e-5) * lng_ref[...] + lnb_ref[...]
    hidden_ref[...] = hidden
    ck_ref[...] = jnp.tanh(_dot(hidden, wkp_ref[...]) + bkp_ref[...])
    cv_ref[...] = jnp.tanh(_dot(hidden, wvp_ref[...]) + bvp_ref[...])
    hg_ref[...] = _dot(hidden, wsm_ref[...]) + bsm_ref[...]


def _cell(cur, prev, gsem, gwork, wsem, wwork, p):
    grid = (B // _BB3,)
    full = lambda r, c: pl.BlockSpec((r, c), lambda i: (0, 0))
    out = pl.pallas_call(
        _cell_body,
        grid=grid,
        in_specs=[
            pl.BlockSpec((_BB3, D), lambda i: (i, 0)),
            pl.BlockSpec((_BB3, D), lambda i: (i, 0)),
            pl.BlockSpec((_BB3, K, D), lambda i: (i, 0, 0)),
            pl.BlockSpec((_BB3, K, D), lambda i: (i, 0, 0)),
            pl.BlockSpec((_BB3, K), lambda i: (i, 0)),
            pl.BlockSpec((_BB3, K), lambda i: (i, 0)),
            full(3 * D, D), full(3 * D, D), full(2 * D, D),
            full(1, D), full(1, D), full(1, D), full(1, D), full(1, D),
            full(D, D), full(D, D), full(1, D), full(1, D),
            full(D, 4), full(1, 4),
        ],
        out_specs=[
            pl.BlockSpec((_BB3, D), lambda i: (i, 0)),
            pl.BlockSpec((_BB3, D), lambda i: (i, 0)),
            pl.BlockSpec((_BB3, D), lambda i: (i, 0)),
            pl.BlockSpec((_BB3, 4), lambda i: (i, 0)),
        ],
        out_shape=[
            jax.ShapeDtypeStruct((B, D), jnp.float32),
            jax.ShapeDtypeStruct((B, D), jnp.float32),
            jax.ShapeDtypeStruct((B, D), jnp.float32),
            jax.ShapeDtypeStruct((B, 4), jnp.float32),
        ],
        compiler_params=pltpu.CompilerParams(
            dimension_semantics=("arbitrary",)),
    )(cur, prev, gsem, gwork, wsem, wwork,
      p['cell_Wg'], p['cell_Wc'], p['cell_Wo'],
      p['cell_bg'][None, :], p['cell_bc'][None, :], p['cell_bo'][None, :],
      p['cell_ln_g'][None, :], p['cell_ln_b'][None, :],
      p['w_Wk'], p['w_Wv'], p['w_bk'][None, :], p['w_bv'][None, :],
      jnp.concatenate([p['w_Wwg'], p['w_Wmg'], p['w_Wbg'], p['w_Wig']], 1),
      jnp.concatenate([p['w_bwg'], p['w_bmg'], p['w_bbg'], p['w_big']])[None, :])
    return out


_BB4 = 32


def _writer_body(ck_ref, cv_ref, hg_ref, wk_ref, wv_ref, wprot_ref,
                 wusage_ref, wage_ref, wsob_ref, bsob_ref,
                 uk_ref, uv_ref, uprot_ref):
    ck = ck_ref[...]
    cv = cv_ref[...]
    wk = wk_ref[...]
    wv = wv_ref[...]
    nck = ck / jnp.maximum(jnp.sqrt(jnp.sum(ck * ck, -1, keepdims=True)), 1e-6)
    wkn = jnp.maximum(jnp.sqrt(jnp.sum(wk * wk, -1)), 1e-6)
    sim = jnp.sum(nck[:, None, :] * wk, -1) / wkn
    wvsq = jnp.sum(wv * wv, -1)
    norm_occ = jnp.clip(jnp.sqrt(wvsq) * (1.0 / (D ** 0.5)), 0.0, 1.0)
    wso = jnp.reshape(wsob_ref[0:1, :], (1, 1, D))
    wsp = jnp.reshape(wsob_ref[1:2, :], (1, 1, D))
    learned_occ = jax.nn.sigmoid(jnp.sum(wv * wso, -1) + bsob_ref[0:1, 0:1])
    learned_prot = jax.nn.sigmoid(jnp.sum(wv * wsp, -1) + bsob_ref[0:1, 1:2])
    occ = jnp.clip(0.5 * learned_occ + 0.5 * norm_occ, 0.0, 1.0)
    eff_prot = jnp.clip(0.4 * learned_prot + 0.6 * wprot_ref[...], 0.0, 1.0)
    eff_usage = jnp.clip(0.5 * occ + 0.5 * wusage_ref[...], 0.0, 1.0)
    eff_age = jnp.clip(wage_ref[...], 0.0, 1.0)
    replace_scores = (1.15 * (1.0 - occ) + 0.85 * (1.0 - eff_prot)
                      + 0.65 * eff_age + 0.45 * (1.0 - eff_usage)
                      + 0.25 * (1.0 - sim))
    iota_n = lax.broadcasted_iota(jnp.int32, (_BB4, N), 1)

    def argmax_low(a):
        m = jnp.max(a, -1, keepdims=True)
        return jnp.min(jnp.where(a == m, iota_n, N), -1, keepdims=True)

    merge_idx = argmax_low(sim)
    replace_idx = argmax_low(replace_scores)
    onehot_m = (iota_n == merge_idx).astype(jnp.float32)
    max_sim = jnp.sum(onehot_m * sim, -1, keepdims=True)
    m_occ = jnp.sum(onehot_m * occ, -1, keepdims=True)
    m_usage = jnp.sum(onehot_m * eff_usage, -1, keepdims=True)
    m_age = jnp.sum(onehot_m * eff_age, -1, keepdims=True)
    write_strength = jax.nn.sigmoid(hg_ref[:, 0:1])
    merge_pref = jax.nn.sigmoid(hg_ref[:, 1:2] + 2.4 * max_sim
                                + 1.6 * (m_occ - 0.5) + 1.0 * (m_usage - 0.5)
                                - 0.8 * m_age)
    binding = jax.nn.sigmoid(hg_ref[:, 2:3] + 2.2 * max_sim)
    importance = jax.nn.sigmoid(hg_ref[:, 3:4])
    use_merge = (merge_pref >= 0.5) & (max_sim > 0.55) & (m_occ > 0.35)
    target_idx = jnp.where(use_merge, merge_idx, replace_idx)
    target_w = (iota_n == target_idx).astype(jnp.float32)
    conflict = jnp.clip(1.0 - sim, 0.0, 1.0)
    overwrite = ((0.15 + 0.85 * write_strength) * target_w
                 * (1.0 - 0.65 * eff_prot * conflict))
    key_mix = jnp.where(use_merge, 0.22 + 0.38 * binding, 0.78 + 0.18 * binding)
    value_mix = jnp.where(use_merge, 0.45 + 0.35 * importance,
                          0.75 + 0.2 * importance)
    ok = (overwrite * key_mix)[:, :, None]
    ov = (overwrite * value_mix)[:, :, None]
    uk_ref[...] = wk + ok * (ck[:, None, :] - wk)
    uv_ref[...] = wv + ov * (cv[:, None, :] - wv)
    boost = overwrite * (0.5 + 0.5 * importance)
    uprot_ref[...] = jnp.clip(wprot_ref[...] * 0.99 + boost, 0.0, 1.0)


def _writer(ck, cv, hg, wk, wv, wprot, wusage, wage, p):
    grid = (B // _BB4,)
    return pl.pallas_call(
        _writer_body,
        grid=grid,
        in_specs=[
            pl.BlockSpec((_BB4, D), lambda i: (i, 0)),
            pl.BlockSpec((_BB4, D), lambda i: (i, 0)),
            pl.BlockSpec((_BB4, 4), lambda i: (i, 0)),
            pl.BlockSpec((_BB4, N, D), lambda i: (i, 0, 0)),
            pl.BlockSpec((_BB4, N, D), lambda i: (i, 0, 0)),
            pl.BlockSpec((_BB4, N), lambda i: (i, 0)),
            pl.BlockSpec((_BB4, N), lambda i: (i, 0)),
            pl.BlockSpec((_BB4, N), lambda i: (i, 0)),
            pl.BlockSpec((2, D), lambda i: (0, 0)),
            pl.BlockSpec((1, 2), lambda i: (0, 0)),
        ],
        out_specs=[
            pl.BlockSpec((_BB4, N, D), lambda i: (i, 0, 0)),
            pl.BlockSpec((_BB4, N, D), lambda i: (i, 0, 0)),
            pl.BlockSpec((_BB4, N), lambda i: (i, 0)),
        ],
        out_shape=[
            jax.ShapeDtypeStruct((B, N, D), jnp.float32),
            jax.ShapeDtypeStruct((B, N, D), jnp.float32),
            jax.ShapeDtypeStruct((B, N), jnp.float32),
        ],
        compiler_params=pltpu.CompilerParams(
            dimension_semantics=("arbitrary",)),
    )(ck, cv, hg, wk, wv, wprot, wusage, wage,
      jnp.concatenate([p['w_Wso'], p['w_Wsp']], 1).T,
      jnp.concatenate([p['w_bso'], p['w_bsp']])[None, :])


def kernel(current, previous, working_keys, working_values, working_protection,
           working_usage, working_age, semantic_keys, semantic_values, params):
    p = params
    sem_idx, widx, w_sem, w_work = _router(
        current, previous, working_keys, semantic_keys, p['router_Wq'])
    g_sem, g_work = _gather_rows(
        sem_idx.reshape(-1), widx.reshape(-1),
        semantic_values, working_values.reshape(B * N, D))
    hidden, ck, cv, hg = _cell(
        current, previous, g_sem.reshape(B, K, D), g_work.reshape(B, K, D),
        w_sem, w_work, p)
    uk, uv, uprot = _writer(
        ck, cv, hg, working_keys, working_values, working_protection,
        working_usage, working_age, p)
    return hidden, uk, uv, uprot


# speedup vs baseline: 16.2221x; 1.8997x over previous
import functools

import jax
import jax.numpy as jnp
from jax import lax
from jax.experimental import pallas as pl
from jax.experimental.pallas import tpu as pltpu
from jax.experimental.pallas import tpu_sc as plsc

D = 1024
N = 32
S = 2048
K = 8
B = 1024

_PREC = lax.Precision.DEFAULT


def _dot(a, b):
    return lax.dot_general(a, b, (((1,), (0,)), ((), ())),
                           precision=_PREC, preferred_element_type=jnp.float32)


def _dot_t(a, b):
    return lax.dot_general(a, b, (((1,), (1,)), ((), ())),
                           precision=_PREC, preferred_element_type=jnp.float32)


_BB1 = 64


def _router_body(cur_ref, prev_ref, wk_ref, sk_ref, wq_ref,
                 semidx_ref, widx_ref, wsem_ref, wwork_ref):
    cur = cur_ref[...]
    prev = prev_ref[...]
    q = _dot(cur, wq_ref[pl.ds(0, D), :]) + _dot(prev, wq_ref[pl.ds(D, D), :])
    q = q / jnp.maximum(jnp.sqrt(jnp.sum(q * q, -1, keepdims=True)), 1e-6)
    sk = sk_ref[...]
    skn = jnp.maximum(jnp.sqrt(jnp.sum(sk * sk, -1, keepdims=True)), 1e-6)
    nsk = sk / skn
    ss = _dot_t(q, nsk)
    ws_list = []
    for n in range(N):
        wkn = wk_ref[:, n, :]
        dp = jnp.sum(q * wkn, -1, keepdims=True)
        nn = jnp.maximum(jnp.sqrt(jnp.sum(wkn * wkn, -1, keepdims=True)), 1e-6)
        ws_list.append(dp / nn)
    ws = jnp.concatenate(ws_list, -1)
    scores = jnp.concatenate([ws, ss], -1)
    T = N + S
    iota = lax.broadcasted_iota(jnp.int32, (_BB1, T), 1)
    s = scores
    tops, topi = [], []
    for _ in range(K):
        m = jnp.max(s, -1, keepdims=True)
        idx = jnp.min(jnp.where(s == m, iota, T), -1, keepdims=True)
        tops.append(m)
        topi.append(idx)
        s = jnp.where(iota == idx, -jnp.inf, s)
    top_s = jnp.concatenate(tops, -1)
    top_i = jnp.concatenate(topi, -1)
    e = jnp.exp(top_s - jnp.max(top_s, -1, keepdims=True))
    w = e / jnp.sum(e, -1, keepdims=True)
    row = pl.program_id(0) * _BB1 + lax.broadcasted_iota(jnp.int32, (_BB1, K), 0)
    semidx_ref[...] = jnp.clip(top_i - N, 0, S - 1)
    widx_ref[...] = row * N + jnp.clip(top_i, 0, N - 1)
    wsem_ref[...] = jnp.where(top_i >= N, w, 0.0)
    wwork_ref[...] = jnp.where(top_i < N, w, 0.0)


def _router(cur, prev, wk, sk, wq):
    grid = (B // _BB1,)
    return pl.pallas_call(
        _router_body,
        grid=grid,
        in_specs=[
            pl.BlockSpec((_BB1, D), lambda i: (i, 0)),
            pl.BlockSpec((_BB1, D), lambda i: (i, 0)),
            pl.BlockSpec((_BB1, N, D), lambda i: (i, 0, 0)),
            pl.BlockSpec((S, D), lambda i: (0, 0)),
            pl.BlockSpec((2 * D, D), lambda i: (0, 0)),
        ],
        out_specs=[
            pl.BlockSpec((_BB1, K), lambda i: (i, 0)),
            pl.BlockSpec((_BB1, K), lambda i: (i, 0)),
            pl.BlockSpec((_BB1, K), lambda i: (i, 0)),
            pl.BlockSpec((_BB1, K), lambda i: (i, 0)),
        ],
        out_shape=[
            jax.ShapeDtypeStruct((B, K), jnp.int32),
            jax.ShapeDtypeStruct((B, K), jnp.int32),
            jax.ShapeDtypeStruct((B, K), jnp.float32),
            jax.ShapeDtypeStruct((B, K), jnp.float32),
        ],
        compiler_params=pltpu.CompilerParams(
            dimension_semantics=("arbitrary",)),
    )(cur, prev, wk, sk, wq)


_ROWS = B * K
_NW = 32
_RPW = _ROWS // _NW
_CH = 32
_NCHUNK = _RPW // _CH


def _gather_rows(sem_idx_flat, widx_flat, sv, wv_flat):
    mesh = plsc.VectorSubcoreMesh(core_axis_name="c", subcore_axis_name="s")

    @functools.partial(
        pl.kernel,
        out_type=(jax.ShapeDtypeStruct((_ROWS, D), jnp.float32),
                  jax.ShapeDtypeStruct((_ROWS, D), jnp.float32)),
        mesh=mesh,
        scratch_types=[
            pltpu.VMEM((_CH,), jnp.int32),
            pltpu.VMEM((_CH, D), jnp.float32),
            pltpu.VMEM((_CH,), jnp.int32),
            pltpu.VMEM((_CH, D), jnp.float32),
            pltpu.SemaphoreType.DMA,
            pltpu.SemaphoreType.DMA,
        ],
    )
    def k(semidx_hbm, widx_hbm, sv_hbm, wvf_hbm, gsem_hbm, gwork_hbm,
          idx1_v, rows1_v, idx2_v, rows2_v, dsem1, dsem2):
        wid = lax.axis_index("s") * 2 + lax.axis_index("c")
        base = wid * _RPW

        def body(i, carry):
            off = base + i * _CH
            pltpu.sync_copy(semidx_hbm.at[pl.ds(off, _CH)], idx1_v)
            pltpu.sync_copy(widx_hbm.at[pl.ds(off, _CH)], idx2_v)
            c1 = pltpu.async_copy(sv_hbm.at[idx1_v], rows1_v, dsem1)
            c2 = pltpu.async_copy(wvf_hbm.at[idx2_v], rows2_v, dsem2)
            c1.wait()
            c2.wait()
            pltpu.sync_copy(rows1_v, gsem_hbm.at[pl.ds(off, _CH)])
            pltpu.sync_copy(rows2_v, gwork_hbm.at[pl.ds(off, _CH)])
            return carry

        lax.fori_loop(0, _NCHUNK, body, 0)

    return k(sem_idx_flat, widx_flat, sv, wv_flat)


_BB3 = 64


def _cell_body(cur_ref, prev_ref, gsem_ref, gwork_ref, wsem_ref, wwork_ref,
               wg_ref, wc_ref, wo_ref, bg_ref, bc_ref, bo_ref, lng_ref, lnb_ref,
               wkp_ref, wvp_ref, bkp_ref, bvp_ref, wsm_ref, bsm_ref,
               hidden_ref, ck_ref, cv_ref, hg_ref):
    cur = cur_ref[...]
    prev = prev_ref[...]
    mr = jnp.zeros_like(cur)
    for k in range(K):
        mr = mr + wsem_ref[:, k:k + 1] * gsem_ref[:, k, :]
        mr = mr + wwork_ref[:, k:k + 1] * gwork_ref[:, k, :]
    gate = jax.nn.sigmoid(_dot(cur, wg_ref[pl.ds(0, D), :])
                          + _dot(prev, wg_ref[pl.ds(D, D), :])
                          + _dot(mr, wg_ref[pl.ds(2 * D, D), :]) + bg_ref[...])
    cand = jnp.tanh(_dot(cur, wc_ref[pl.ds(0, D), :])
                    + _dot(prev, wc_ref[pl.ds(D, D), :])
                    + _dot(mr, wc_ref[pl.ds(2 * D, D), :]) + bc_ref[...])
    h = (1.0 - gate) * prev + gate * cand
    x = _dot(cur, wo_ref[pl.ds(0, D), :]) + _dot(h, wo_ref[pl.ds(D, D), :]) \
        + bo_ref[...] + cur
    mu = jnp.mean(x, -1, keepdims=True)
    xc = x - mu
    var = jnp.mean(xc * xc, -1, keepdims=True)
    hidden = xc / jnp.sqrt(var + 1e-5) * lng_ref[...] + lnb_ref[...]
    hidden_ref[...] = hidden
    ck_ref[...] = jnp.tanh(_dot(hidden, wkp_ref[...]) + bkp_ref[...])
    cv_ref[...] = jnp.tanh(_dot(hidden, wvp_ref[...]) + bvp_ref[...])
    hg_ref[...] = _dot(hidden, wsm_ref[...]) + bsm_ref[...]


def _cell(cur, prev, gsem, gwork, wsem, wwork, p):
    grid = (B // _BB3,)
    full = lambda r, c: pl.BlockSpec((r, c), lambda i: (0, 0))
    out = pl.pallas_call(
        _cell_body,
        grid=grid,
        in_specs=[
            pl.BlockSpec((_BB3, D), lambda i: (i, 0)),
            pl.BlockSpec((_BB3, D), lambda i: (i, 0)),
            pl.BlockSpec((_BB3, K, D), lambda i: (i, 0, 0)),
            pl.BlockSpec((_BB3, K, D), lambda i: (i, 0, 0)),
            pl.BlockSpec((_BB3, K), lambda i: (i, 0)),
            pl.BlockSpec((_BB3, K), lambda i: (i, 0)),
            full(3 * D, D), full(3 * D, D), full(2 * D, D),
            full(1, D), full(1, D), full(1, D), full(1, D), full(1, D),
            full(D, D), full(D, D), full(1, D), full(1, D),
            full(D, 4), full(1, 4),
        ],
        out_specs=[
            pl.BlockSpec((_BB3, D), lambda i: (i, 0)),
            pl.BlockSpec((_BB3, D), lambda i: (i, 0)),
            pl.BlockSpec((_BB3, D), lambda i: (i, 0)),
            pl.BlockSpec((_BB3, 4), lambda i: (i, 0)),
        ],
        out_shape=[
            jax.ShapeDtypeStruct((B, D), jnp.float32),
            jax.ShapeDtypeStruct((B, D), jnp.float32),
            jax.ShapeDtypeStruct((B, D), jnp.float32),
            jax.ShapeDtypeStruct((B, 4), jnp.float32),
        ],
        compiler_params=pltpu.CompilerParams(
            dimension_semantics=("arbitrary",)),
    )(cur, prev, gsem, gwork, wsem, wwork,
      p['cell_Wg'], p['cell_Wc'], p['cell_Wo'],
      p['cell_bg'][None, :], p['cell_bc'][None, :], p['cell_bo'][None, :],
      p['cell_ln_g'][None, :], p['cell_ln_b'][None, :],
      p['w_Wk'], p['w_Wv'], p['w_bk'][None, :], p['w_bv'][None, :],
      jnp.concatenate([p['w_Wwg'], p['w_Wmg'], p['w_Wbg'], p['w_Wig']], 1),
      jnp.concatenate([p['w_bwg'], p['w_bmg'], p['w_bbg'], p['w_big']])[None, :])
    return out


_BB4 = 32


def _writer_body(ck_ref, cv_ref, hg_ref, wk_ref, wv_ref, wprot_ref,
                 wusage_ref, wage_ref, wsob_ref, bsob_ref,
                 uk_ref, uv_ref, uprot_ref):
    ck = ck_ref[...]
    cv = cv_ref[...]
    wk = wk_ref[...]
    wv = wv_ref[...]
    nck = ck / jnp.maximum(jnp.sqrt(jnp.sum(ck * ck, -1, keepdims=True)), 1e-6)
    wkn = jnp.maximum(jnp.sqrt(jnp.sum(wk * wk, -1)), 1e-6)
    sim = jnp.sum(nck[:, None, :] * wk, -1) / wkn
    wvsq = jnp.sum(wv * wv, -1)
    norm_occ = jnp.clip(jnp.sqrt(wvsq) * (1.0 / (D ** 0.5)), 0.0, 1.0)
    wso = jnp.reshape(wsob_ref[0:1, :], (1, 1, D))
    wsp = jnp.reshape(wsob_ref[1:2, :], (1, 1, D))
    learned_occ = jax.nn.sigmoid(jnp.sum(wv * wso, -1) + bsob_ref[0:1, 0:1])
    learned_prot = jax.nn.sigmoid(jnp.sum(wv * wsp, -1) + bsob_ref[0:1, 1:2])
    occ = jnp.clip(0.5 * learned_occ + 0.5 * norm_occ, 0.0, 1.0)
    eff_prot = jnp.clip(0.4 * learned_prot + 0.6 * wprot_ref[...], 0.0, 1.0)
    eff_usage = jnp.clip(0.5 * occ + 0.5 * wusage_ref[...], 0.0, 1.0)
    eff_age = jnp.clip(wage_ref[...], 0.0, 1.0)
    replace_scores = (1.15 * (1.0 - occ) + 0.85 * (1.0 - eff_prot)
                      + 0.65 * eff_age + 0.45 * (1.0 - eff_usage)
                      + 0.25 * (1.0 - sim))
    iota_n = lax.broadcasted_iota(jnp.int32, (_BB4, N), 1)

    def argmax_low(a):
        m = jnp.max(a, -1, keepdims=True)
        return jnp.min(jnp.where(a == m, iota_n, N), -1, keepdims=True)

    merge_idx = argmax_low(sim)
    replace_idx = argmax_low(replace_scores)
    onehot_m = (iota_n == merge_idx).astype(jnp.float32)
    max_sim = jnp.sum(onehot_m * sim, -1, keepdims=True)
    m_occ = jnp.sum(onehot_m * occ, -1, keepdims=True)
    m_usage = jnp.sum(onehot_m * eff_usage, -1, keepdims=True)
    m_age = jnp.sum(onehot_m * eff_age, -1, keepdims=True)
    write_strength = jax.nn.sigmoid(hg_ref[:, 0:1])
    merge_pref = jax.nn.sigmoid(hg_ref[:, 1:2] + 2.4 * max_sim
                                + 1.6 * (m_occ - 0.5) + 1.0 * (m_usage - 0.5)
                                - 0.8 * m_age)
    binding = jax.nn.sigmoid(hg_ref[:, 2:3] + 2.2 * max_sim)
    importance = jax.nn.sigmoid(hg_ref[:, 3:4])
    use_merge = (merge_pref >= 0.5) & (max_sim > 0.55) & (m_occ > 0.35)
    target_idx = jnp.where(use_merge, merge_idx, replace_idx)
    target_w = (iota_n == target_idx).astype(jnp.float32)
    conflict = jnp.clip(1.0 - sim, 0.0, 1.0)
    overwrite = ((0.15 + 0.85 * write_strength) * target_w
                 * (1.0 - 0.65 * eff_prot * conflict))
    key_mix = jnp.where(use_merge, 0.22 + 0.38 * binding, 0.78 + 0.18 * binding)
    value_mix = jnp.where(use_merge, 0.45 + 0.35 * importance,
                          0.75 + 0.2 * importance)
    ok = (overwrite * key_mix)[:, :, None]
    ov = (overwrite * value_mix)[:, :, None]
    uk_ref[...] = wk + ok * (ck[:, None, :] - wk)
    uv_ref[...] = wv + ov * (cv[:, None, :] - wv)
    boost = overwrite * (0.5 + 0.5 * importance)
    uprot_ref[...] = jnp.clip(wprot_ref[...] * 0.99 + boost, 0.0, 1.0)


def _writer(ck, cv, hg, wk, wv, wprot, wusage, wage, p):
    grid = (B // _BB4,)
    return pl.pallas_call(
        _writer_body,
        grid=grid,
        in_specs=[
            pl.BlockSpec((_BB4, D), lambda i: (i, 0)),
            pl.BlockSpec((_BB4, D), lambda i: (i, 0)),
            pl.BlockSpec((_BB4, 4), lambda i: (i, 0)),
            pl.BlockSpec((_BB4, N, D), lambda i: (i, 0, 0)),
            pl.BlockSpec((_BB4, N, D), lambda i: (i, 0, 0)),
            pl.BlockSpec((_BB4, N), lambda i: (i, 0)),
            pl.BlockSpec((_BB4, N), lambda i: (i, 0)),
            pl.BlockSpec((_BB4, N), lambda i: (i, 0)),
            pl.BlockSpec((2, D), lambda i: (0, 0)),
            pl.BlockSpec((1, 2), lambda i: (0, 0)),
        ],
        out_specs=[
            pl.BlockSpec((_BB4, N, D), lambda i: (i, 0, 0)),
            pl.BlockSpec((_BB4, N, D), lambda i: (i, 0, 0)),
            pl.BlockSpec((_BB4, N), lambda i: (i, 0)),
        ],
        out_shape=[
            jax.ShapeDtypeStruct((B, N, D), jnp.float32),
            jax.ShapeDtypeStruct((B, N, D), jnp.float32),
            jax.ShapeDtypeStruct((B, N), jnp.float32),
        ],
        compiler_params=pltpu.CompilerParams(
            dimension_semantics=("arbitrary",)),
    )(ck, cv, hg, wk, wv, wprot, wusage, wage,
      jnp.concatenate([p['w_Wso'], p['w_Wsp']], 1).T,
      jnp.concatenate([p['w_bso'], p['w_bsp']])[None, :])


def kernel(current, previous, working_keys, working_values, working_protection,
           working_usage, working_age, semantic_keys, semantic_values, params):
    p = params
    sem_idx, widx, w_sem, w_work = _router(
        current, previous, working_keys, semantic_keys, p['router_Wq'])
    g_sem, g_work = _gather_rows(
        sem_idx.reshape(-1), widx.reshape(-1),
        semantic_values, working_values.reshape(B * N, D))
    hidden, ck, cv, hg = _cell(
        current, previous, g_sem.reshape(B, K, D), g_work.reshape(B, K, D),
        w_sem, w_work, p)
    uk, uv, uprot = _writer(
        ck, cv, hg, working_keys, working_values, working_protection,
        working_usage, working_age, p)
    return hidden, uk, uv, uprot


# speedup vs baseline: 16.3303x; 1.0067x over previous
import functools

import jax
import jax.numpy as jnp
from jax import lax
from jax.experimental import pallas as pl
from jax.experimental.pallas import tpu as pltpu
from jax.experimental.pallas import tpu_sc as plsc

D = 1024
N = 32
S = 2048
K = 8
B = 1024

_PREC = lax.Precision.DEFAULT


def _dot(a, b):
    return lax.dot_general(a, b, (((1,), (0,)), ((), ())),
                           precision=_PREC, preferred_element_type=jnp.float32)


def _dot_t(a, b):
    return lax.dot_general(a, b, (((1,), (1,)), ((), ())),
                           precision=_PREC, preferred_element_type=jnp.float32)


_BB1 = 64


def _router_body(cur_ref, prev_ref, wk_ref, sk_ref, wq_ref,
                 semidx_ref, widx_ref, wsem_ref, wwork_ref):
    cur = cur_ref[...]
    prev = prev_ref[...]
    q = _dot(cur, wq_ref[pl.ds(0, D), :]) + _dot(prev, wq_ref[pl.ds(D, D), :])
    q = q / jnp.maximum(jnp.sqrt(jnp.sum(q * q, -1, keepdims=True)), 1e-6)
    sk = sk_ref[...]
    skn = jnp.maximum(jnp.sqrt(jnp.sum(sk * sk, -1, keepdims=True)), 1e-6)
    nsk = sk / skn
    ss = _dot_t(q, nsk)
    ws_list = []
    for n in range(N):
        wkn = wk_ref[:, n, :]
        dp = jnp.sum(q * wkn, -1, keepdims=True)
        nn = jnp.maximum(jnp.sqrt(jnp.sum(wkn * wkn, -1, keepdims=True)), 1e-6)
        ws_list.append(dp / nn)
    ws = jnp.concatenate(ws_list, -1)
    scores = jnp.concatenate([ws, ss], -1)
    T = N + S
    iota = lax.broadcasted_iota(jnp.int32, (_BB1, T), 1)
    s = scores
    tops, topi = [], []
    for _ in range(K):
        m = jnp.max(s, -1, keepdims=True)
        idx = jnp.min(jnp.where(s == m, iota, T), -1, keepdims=True)
        tops.append(m)
        topi.append(idx)
        s = jnp.where(iota == idx, -jnp.inf, s)
    top_s = jnp.concatenate(tops, -1)
    top_i = jnp.concatenate(topi, -1)
    e = jnp.exp(top_s - jnp.max(top_s, -1, keepdims=True))
    w = e / jnp.sum(e, -1, keepdims=True)
    row = pl.program_id(0) * _BB1 + lax.broadcasted_iota(jnp.int32, (_BB1, K), 0)
    semidx_ref[...] = jnp.clip(top_i - N, 0, S - 1)
    widx_ref[...] = row * N + jnp.clip(top_i, 0, N - 1)
    wsem_ref[...] = jnp.where(top_i >= N, w, 0.0)
    wwork_ref[...] = jnp.where(top_i < N, w, 0.0)


def _router(cur, prev, wk, sk, wq):
    grid = (B // _BB1,)
    return pl.pallas_call(
        _router_body,
        grid=grid,
        in_specs=[
            pl.BlockSpec((_BB1, D), lambda i: (i, 0)),
            pl.BlockSpec((_BB1, D), lambda i: (i, 0)),
            pl.BlockSpec((_BB1, N, D), lambda i: (i, 0, 0)),
            pl.BlockSpec((S, D), lambda i: (0, 0)),
            pl.BlockSpec((2 * D, D), lambda i: (0, 0)),
        ],
        out_specs=[
            pl.BlockSpec((_BB1, K), lambda i: (i, 0)),
            pl.BlockSpec((_BB1, K), lambda i: (i, 0)),
            pl.BlockSpec((_BB1, K), lambda i: (i, 0)),
            pl.BlockSpec((_BB1, K), lambda i: (i, 0)),
        ],
        out_shape=[
            jax.ShapeDtypeStruct((B, K), jnp.int32),
            jax.ShapeDtypeStruct((B, K), jnp.int32),
            jax.ShapeDtypeStruct((B, K), jnp.float32),
            jax.ShapeDtypeStruct((B, K), jnp.float32),
        ],
        compiler_params=pltpu.CompilerParams(
            dimension_semantics=("arbitrary",)),
    )(cur, prev, wk, sk, wq)


_ROWS = B * K
_NW = 32
_RPW = _ROWS // _NW
_CH = 32
_NCHUNK = _RPW // _CH


def _gather_rows(sem_idx_flat, widx_flat, sv, wv_flat):
    mesh = plsc.VectorSubcoreMesh(core_axis_name="c", subcore_axis_name="s")

    @functools.partial(
        pl.kernel,
        out_type=(jax.ShapeDtypeStruct((_ROWS, D), jnp.float32),
                  jax.ShapeDtypeStruct((_ROWS, D), jnp.float32)),
        mesh=mesh,
        scratch_types=[
            pltpu.VMEM((_CH,), jnp.int32),
            pltpu.VMEM((_CH, D), jnp.float32),
            pltpu.VMEM((_CH,), jnp.int32),
            pltpu.VMEM((_CH, D), jnp.float32),
            pltpu.SemaphoreType.DMA,
            pltpu.SemaphoreType.DMA,
        ],
    )
    def k(semidx_hbm, widx_hbm, sv_hbm, wvf_hbm, gsem_hbm, gwork_hbm,
          idx1_v, rows1_v, idx2_v, rows2_v, dsem1, dsem2):
        wid = lax.axis_index("s") * 2 + lax.axis_index("c")
        base = wid * _RPW

        def body(i, carry):
            off = base + i * _CH
            pltpu.sync_copy(semidx_hbm.at[pl.ds(off, _CH)], idx1_v)
            pltpu.sync_copy(widx_hbm.at[pl.ds(off, _CH)], idx2_v)
            c1 = pltpu.async_copy(sv_hbm.at[idx1_v], rows1_v, dsem1)
            c2 = pltpu.async_copy(wvf_hbm.at[idx2_v], rows2_v, dsem2)
            c1.wait()
            c2.wait()
            pltpu.sync_copy(rows1_v, gsem_hbm.at[pl.ds(off, _CH)])
            pltpu.sync_copy(rows2_v, gwork_hbm.at[pl.ds(off, _CH)])
            return carry

        lax.fori_loop(0, _NCHUNK, body, 0)

    return k(sem_idx_flat, widx_flat, sv, wv_flat)


_BB3 = 64


def _cell_pre_body(cur_ref, prev_ref, wg_ref, wc_ref, wo_ref,
                   bg_ref, bc_ref, bo_ref,
                   pg_ref, pc_ref, po_ref):
    cur = cur_ref[...]
    prev = prev_ref[...]
    pg_ref[...] = _dot(cur, wg_ref[pl.ds(0, D), :]) \
        + _dot(prev, wg_ref[pl.ds(D, D), :]) + bg_ref[...]
    pc_ref[...] = _dot(cur, wc_ref[pl.ds(0, D), :]) \
        + _dot(prev, wc_ref[pl.ds(D, D), :]) + bc_ref[...]
    po_ref[...] = _dot(cur, wo_ref[pl.ds(0, D), :]) + bo_ref[...] + cur


def _cell_pre(cur, prev, p):
    grid = (B // _BB3,)
    full = lambda r, c: pl.BlockSpec((r, c), lambda i: (0, 0))
    return pl.pallas_call(
        _cell_pre_body,
        grid=grid,
        in_specs=[
            pl.BlockSpec((_BB3, D), lambda i: (i, 0)),
            pl.BlockSpec((_BB3, D), lambda i: (i, 0)),
            full(3 * D, D), full(3 * D, D), full(2 * D, D),
            full(1, D), full(1, D), full(1, D),
        ],
        out_specs=[
            pl.BlockSpec((_BB3, D), lambda i: (i, 0)),
            pl.BlockSpec((_BB3, D), lambda i: (i, 0)),
            pl.BlockSpec((_BB3, D), lambda i: (i, 0)),
        ],
        out_shape=[
            jax.ShapeDtypeStruct((B, D), jnp.float32),
            jax.ShapeDtypeStruct((B, D), jnp.float32),
            jax.ShapeDtypeStruct((B, D), jnp.float32),
        ],
        compiler_params=pltpu.CompilerParams(
            dimension_semantics=("arbitrary",)),
    )(cur, prev, p['cell_Wg'], p['cell_Wc'], p['cell_Wo'],
      p['cell_bg'][None, :], p['cell_bc'][None, :], p['cell_bo'][None, :])


def _cell_body(prev_ref, gsem_ref, gwork_ref, wsem_ref, wwork_ref,
               pg_ref, pc_ref, po_ref,
               wg_ref, wc_ref, wo_ref, lng_ref, lnb_ref,
               wkp_ref, wvp_ref, bkp_ref, bvp_ref, wsm_ref, bsm_ref,
               hidden_ref, ck_ref, cv_ref, hg_ref):
    prev = prev_ref[...]
    mr = wsem_ref[:, 0:1] * gsem_ref[:, 0, :] + wwork_ref[:, 0:1] * gwork_ref[:, 0, :]
    for k in range(1, K):
        mr = mr + wsem_ref[:, k:k + 1] * gsem_ref[:, k, :]
        mr = mr + wwork_ref[:, k:k + 1] * gwork_ref[:, k, :]
    gate = jax.nn.sigmoid(pg_ref[...] + _dot(mr, wg_ref[...]))
    cand = jnp.tanh(pc_ref[...] + _dot(mr, wc_ref[...]))
    h = (1.0 - gate) * prev + gate * cand
    x = po_ref[...] + _dot(h, wo_ref[...])
    mu = jnp.mean(x, -1, keepdims=True)
    xc = x - mu
    var = jnp.mean(xc * xc, -1, keepdims=True)
    hidden = xc / jnp.sqrt(var + 1e-5) * lng_ref[...] + lnb_ref[...]
    hidden_ref[...] = hidden
    ck_ref[...] = jnp.tanh(_dot(hidden, wkp_ref[...]) + bkp_ref[...])
    cv_ref[...] = jnp.tanh(_dot(hidden, wvp_ref[...]) + bvp_ref[...])
    hg_ref[...] = _dot(hidden, wsm_ref[...]) + bsm_ref[...]


def _cell(prev, gsem, gwork, wsem, wwork, pg, pc, po, p):
    grid = (B // _BB3,)
    full = lambda r, c: pl.BlockSpec((r, c), lambda i: (0, 0))
    out = pl.pallas_call(
        _cell_body,
        grid=grid,
        in_specs=[
            pl.BlockSpec((_BB3, D), lambda i: (i, 0)),
            pl.BlockSpec((_BB3, K, D), lambda i: (i, 0, 0)),
            pl.BlockSpec((_BB3, K, D), lambda i: (i, 0, 0)),
            pl.BlockSpec((_BB3, K), lambda i: (i, 0)),
            pl.BlockSpec((_BB3, K), lambda i: (i, 0)),
            pl.BlockSpec((_BB3, D), lambda i: (i, 0)),
            pl.BlockSpec((_BB3, D), lambda i: (i, 0)),
            pl.BlockSpec((_BB3, D), lambda i: (i, 0)),
            full(D, D), full(D, D), full(D, D), full(1, D), full(1, D),
            full(D, D), full(D, D), full(1, D), full(1, D),
            full(D, 4), full(1, 4),
        ],
        out_specs=[
            pl.BlockSpec((_BB3, D), lambda i: (i, 0)),
            pl.BlockSpec((_BB3, D), lambda i: (i, 0)),
            pl.BlockSpec((_BB3, D), lambda i: (i, 0)),
            pl.BlockSpec((_BB3, 4), lambda i: (i, 0)),
        ],
        out_shape=[
            jax.ShapeDtypeStruct((B, D), jnp.float32),
            jax.ShapeDtypeStruct((B, D), jnp.float32),
            jax.ShapeDtypeStruct((B, D), jnp.float32),
            jax.ShapeDtypeStruct((B, 4), jnp.float32),
        ],
        compiler_params=pltpu.CompilerParams(
            dimension_semantics=("arbitrary",)),
    )(prev, gsem, gwork, wsem, wwork, pg, pc, po,
      lax.slice(p['cell_Wg'], (2 * D, 0), (3 * D, D)),
      lax.slice(p['cell_Wc'], (2 * D, 0), (3 * D, D)),
      lax.slice(p['cell_Wo'], (D, 0), (2 * D, D)),
      p['cell_ln_g'][None, :], p['cell_ln_b'][None, :],
      p['w_Wk'], p['w_Wv'], p['w_bk'][None, :], p['w_bv'][None, :],
      jnp.concatenate([p['w_Wwg'], p['w_Wmg'], p['w_Wbg'], p['w_Wig']], 1),
      jnp.concatenate([p['w_bwg'], p['w_bmg'], p['w_bbg'], p['w_big']])[None, :])
    return out


_BB4 = 32


def _writer_body(ck_ref, cv_ref, hg_ref, wk_ref, wv_ref, wprot_ref,
                 wusage_ref, wage_ref, wsob_ref, bsob_ref,
                 uk_ref, uv_ref, uprot_ref):
    ck = ck_ref[...]
    cv = cv_ref[...]
    wk = wk_ref[...]
    wv = wv_ref[...]
    nck = ck / jnp.maximum(jnp.sqrt(jnp.sum(ck * ck, -1, keepdims=True)), 1e-6)
    wkn = jnp.maximum(jnp.sqrt(jnp.sum(wk * wk, -1)), 1e-6)
    sim = jnp.sum(nck[:, None, :] * wk, -1) / wkn
    wvsq = jnp.sum(wv * wv, -1)
    norm_occ = jnp.clip(jnp.sqrt(wvsq) * (1.0 / (D ** 0.5)), 0.0, 1.0)
    wso = jnp.reshape(wsob_ref[0:1, :], (1, 1, D))
    wsp = jnp.reshape(wsob_ref[1:2, :], (1, 1, D))
    learned_occ = jax.nn.sigmoid(jnp.sum(wv * wso, -1) + bsob_ref[0:1, 0:1])
    learned_prot = jax.nn.sigmoid(jnp.sum(wv * wsp, -1) + bsob_ref[0:1, 1:2])
    occ = jnp.clip(0.5 * learned_occ + 0.5 * norm_occ, 0.0, 1.0)
    eff_prot = jnp.clip(0.4 * learned_prot + 0.6 * wprot_ref[...], 0.0, 1.0)
    eff_usage = jnp.clip(0.5 * occ + 0.5 * wusage_ref[...], 0.0, 1.0)
    eff_age = jnp.clip(wage_ref[...], 0.0, 1.0)
    replace_scores = (1.15 * (1.0 - occ) + 0.85 * (1.0 - eff_prot)
                      + 0.65 * eff_age + 0.45 * (1.0 - eff_usage)
                      + 0.25 * (1.0 - sim))
    iota_n = lax.broadcasted_iota(jnp.int32, (_BB4, N), 1)

    def argmax_low(a):
        m = jnp.max(a, -1, keepdims=True)
        return jnp.min(jnp.where(a == m, iota_n, N), -1, keepdims=True)

    merge_idx = argmax_low(sim)
    replace_idx = argmax_low(replace_scores)
    onehot_m = (iota_n == merge_idx).astype(jnp.float32)
    max_sim = jnp.sum(onehot_m * sim, -1, keepdims=True)
    m_occ = jnp.sum(onehot_m * occ, -1, keepdims=True)
    m_usage = jnp.sum(onehot_m * eff_usage, -1, keepdims=True)
    m_age = jnp.sum(onehot_m * eff_age, -1, keepdims=True)
    write_strength = jax.nn.sigmoid(hg_ref[:, 0:1])
    merge_pref = jax.nn.sigmoid(hg_ref[:, 1:2] + 2.4 * max_sim
                                + 1.6 * (m_occ - 0.5) + 1.0 * (m_usage - 0.5)
                                - 0.8 * m_age)
    binding = jax.nn.sigmoid(hg_ref[:, 2:3] + 2.2 * max_sim)
    importance = jax.nn.sigmoid(hg_ref[:, 3:4])
    use_merge = (merge_pref >= 0.5) & (max_sim > 0.55) & (m_occ > 0.35)
    target_idx = jnp.where(use_merge, merge_idx, replace_idx)
    target_w = (iota_n == target_idx).astype(jnp.float32)
    conflict = jnp.clip(1.0 - sim, 0.0, 1.0)
    overwrite = ((0.15 + 0.85 * write_strength) * target_w
                 * (1.0 - 0.65 * eff_prot * conflict))
    key_mix = jnp.where(use_merge, 0.22 + 0.38 * binding, 0.78 + 0.18 * binding)
    value_mix = jnp.where(use_merge, 0.45 + 0.35 * importance,
                          0.75 + 0.2 * importance)
    ok = (overwrite * key_mix)[:, :, None]
    ov = (overwrite * value_mix)[:, :, None]
    uk_ref[...] = wk + ok * (ck[:, None, :] - wk)
    uv_ref[...] = wv + ov * (cv[:, None, :] - wv)
    boost = overwrite * (0.5 + 0.5 * importance)
    uprot_ref[...] = jnp.clip(wprot_ref[...] * 0.99 + boost, 0.0, 1.0)


def _writer(ck, cv, hg, wk, wv, wprot, wusage, wage, p):
    grid = (B // _BB4,)
    return pl.pallas_call(
        _writer_body,
        grid=grid,
        in_specs=[
            pl.BlockSpec((_BB4, D), lambda i: (i, 0)),
            pl.BlockSpec((_BB4, D), lambda i: (i, 0)),
            pl.BlockSpec((_BB4, 4), lambda i: (i, 0)),
            pl.BlockSpec((_BB4, N, D), lambda i: (i, 0, 0)),
            pl.BlockSpec((_BB4, N, D), lambda i: (i, 0, 0)),
            pl.BlockSpec((_BB4, N), lambda i: (i, 0)),
            pl.BlockSpec((_BB4, N), lambda i: (i, 0)),
            pl.BlockSpec((_BB4, N), lambda i: (i, 0)),
            pl.BlockSpec((2, D), lambda i: (0, 0)),
            pl.BlockSpec((1, 2), lambda i: (0, 0)),
        ],
        out_specs=[
            pl.BlockSpec((_BB4, N, D), lambda i: (i, 0, 0)),
            pl.BlockSpec((_BB4, N, D), lambda i: (i, 0, 0)),
            pl.BlockSpec((_BB4, N), lambda i: (i, 0)),
        ],
        out_shape=[
            jax.ShapeDtypeStruct((B, N, D), jnp.float32),
            jax.ShapeDtypeStruct((B, N, D), jnp.float32),
            jax.ShapeDtypeStruct((B, N), jnp.float32),
        ],
        compiler_params=pltpu.CompilerParams(
            dimension_semantics=("arbitrary",)),
    )(ck, cv, hg, wk, wv, wprot, wusage, wage,
      jnp.concatenate([p['w_Wso'], p['w_Wsp']], 1).T,
      jnp.concatenate([p['w_bso'], p['w_bsp']])[None, :])


def kernel(current, previous, working_keys, working_values, working_protection,
           working_usage, working_age, semantic_keys, semantic_values, params):
    p = params
    sem_idx, widx, w_sem, w_work = _router(
        current, previous, working_keys, semantic_keys, p['router_Wq'])
    g_sem, g_work = _gather_rows(
        sem_idx.reshape(-1), widx.reshape(-1),
        semantic_values, working_values.reshape(B * N, D))
    pg, pc, po = _cell_pre(current, previous, p)
    hidden, ck, cv, hg = _cell(
        previous, g_sem.reshape(B, K, D), g_work.reshape(B, K, D),
        w_sem, w_work, pg, pc, po, p)
    uk, uv, uprot = _writer(
        ck, cv, hg, working_keys, working_values, working_protection,
        working_usage, working_age, p)
    return hidden, uk, uv, uprot
